# Initial kernel scaffold; baseline (speedup 1.0000x reference)
import jax
import jax.numpy as jnp
from jax.experimental import pallas as pl


def kernel(x, w1, b1, w2p, b2p):
    raise NotImplementedError("write your pallas kernel here")


# speedup vs baseline: 1.2139x; 1.2139x over previous
import jax
import jax.numpy as jnp
from jax.experimental import pallas as pl
from jax.experimental.pallas import tpu as pltpu

_HIDDEN = 128
_N_ACTION = 2


def _mlp_kernel(x_ref, w1_ref, b1_ref, w2_ref, b2_ref, o_ref):
    h = jnp.dot(x_ref[...], w1_ref[...], preferred_element_type=jnp.float32)
    h = jnp.maximum(h + b1_ref[...], 0.0)
    y = jnp.dot(h, w2_ref[...], preferred_element_type=jnp.float32)
    o_ref[...] = jnp.tanh(y + b2_ref[...])


def kernel(x, w1, b1, w2p, b2p):
    B, n_states = x.shape
    w2 = w2p[:, :_N_ACTION]
    b2 = b2p[:, :_N_ACTION]

    block_b = 8192
    if B <= block_b:
        return pl.pallas_call(
            _mlp_kernel,
            out_shape=jax.ShapeDtypeStruct((B, _N_ACTION), jnp.float32),
        )(x, w1, b1, w2, b2)

    nb = pl.cdiv(B, block_b)
    return pl.pallas_call(
        _mlp_kernel,
        out_shape=jax.ShapeDtypeStruct((B, _N_ACTION), jnp.float32),
        grid=(nb,),
        in_specs=[
            pl.BlockSpec((block_b, n_states), lambda i: (i, 0)),
            pl.BlockSpec((n_states, _HIDDEN), lambda i: (0, 0)),
            pl.BlockSpec((1, _HIDDEN), lambda i: (0, 0)),
            pl.BlockSpec((_HIDDEN, _N_ACTION), lambda i: (0, 0)),
            pl.BlockSpec((1, _N_ACTION), lambda i: (0, 0)),
        ],
        out_specs=pl.BlockSpec((block_b, _N_ACTION), lambda i: (i, 0)),
        compiler_params=pltpu.CompilerParams(
            dimension_semantics=("parallel",)),
    )(x, w1, b1, w2, b2)
